# Initial kernel scaffold; baseline (speedup 1.0000x reference)
import functools

import jax
import jax.numpy as jnp
from jax import lax
from jax.experimental import pallas as pl
from jax.experimental.pallas import tpu as pltpu
from jax.experimental.pallas import tpu_sc as plsc

N_NODES = 10000
N_EDGES = 320000
D_IN = 128
D_HID = 128
D_OUT = 40
D_OUT_PAD = 64

NC = 2
NS = 16
NW = NC * NS
EPW = N_EDGES // NW
CH = 80
CPT = EPW // CH

NP = 10240
RPT = NP // NS

BLK = 1024

_mesh = plsc.VectorSubcoreMesh(core_axis_name="c", subcore_axis_name="s")


def _wid():
    return lax.axis_index("c") * NS + lax.axis_index("s")


@functools.partial(
    pl.kernel,
    out_type=jax.ShapeDtypeStruct((NC, NP), jnp.float32),
    mesh=_mesh,
    scratch_types=[
        pltpu.VMEM((CPT, CH), jnp.int32),
        pltpu.VMEM((CH,), jnp.float32),
        pltpu.VMEM_SHARED((NP,), jnp.float32),
    ],
)
def _deg_kernel(col_hbm, ones_hbm, out_hbm, colv, ones_v, acc):
    cid = lax.axis_index("c")
    sid = lax.axis_index("s")
    pltpu.sync_copy(col_hbm.at[_wid()], colv)
    pltpu.sync_copy(ones_hbm.at[pl.ds(0, CH)], ones_v)
    pltpu.sync_copy(ones_hbm.at[pl.ds(sid * RPT, RPT)],
                    acc.at[pl.ds(sid * RPT, RPT)])
    plsc.subcore_barrier()

    def body(j, carry):
        pltpu.sync_copy(ones_v, acc.at[colv.at[j]], add=True)
        return carry

    lax.fori_loop(0, CPT, body, 0, unroll=False)
    plsc.subcore_barrier()
    pltpu.sync_copy(acc.at[pl.ds(sid * RPT, RPT)],
                    out_hbm.at[cid].at[pl.ds(sid * RPT, RPT)])


def _make_agg(d):
    @functools.partial(
        pl.kernel,
        out_type=jax.ShapeDtypeStruct((NC, NP, d), jnp.float32),
        mesh=_mesh,
        scratch_types=[
            pltpu.VMEM((CPT, CH), jnp.int32),
            pltpu.VMEM((CPT, CH), jnp.int32),
            pltpu.VMEM((CH, d), jnp.float32),
            pltpu.VMEM_SHARED((NP, d), jnp.float32),
        ],
    )
    def agg(x_hbm, row_hbm, col_hbm, out_hbm, rowv, colv, buf, acc):
        cid = lax.axis_index("c")
        sid = lax.axis_index("s")
        pltpu.sync_copy(row_hbm.at[_wid()], rowv)
        pltpu.sync_copy(col_hbm.at[_wid()], colv)
        pltpu.sync_copy(x_hbm.at[pl.ds(sid * RPT, RPT)],
                        acc.at[pl.ds(sid * RPT, RPT)])
        plsc.subcore_barrier()

        def body(j, carry):
            pltpu.sync_copy(x_hbm.at[rowv.at[j]], buf)
            pltpu.sync_copy(buf, acc.at[colv.at[j]], add=True)
            return carry

        lax.fori_loop(0, CPT, body, 0, unroll=False)
        plsc.subcore_barrier()
        pltpu.sync_copy(acc.at[pl.ds(sid * RPT, RPT)],
                        out_hbm.at[cid].at[pl.ds(sid * RPT, RPT)])

    return agg


_agg1 = _make_agg(D_HID)
_agg2 = _make_agg(D_OUT_PAD)


def _tc1_body(deg_ref, x_ref, w1_ref, x1_ref, dv_ref):
    s = deg_ref[:, 0:1] + deg_ref[:, 1:2] - 1.0
    dv = lax.rsqrt(s)
    dv_ref[...] = dv
    x1_ref[...] = jnp.dot(x_ref[...] * dv, w1_ref[...],
                          preferred_element_type=jnp.float32)


def _tc2_body(p_ref, x1_ref, dv_ref, b1_ref, w2_ref, x2_ref):
    x1 = x1_ref[...]
    s = p_ref[0] + p_ref[1] - x1
    dv = dv_ref[...]
    h = jnp.maximum(dv * s + b1_ref[...], 0.0)
    x2_ref[...] = jnp.dot(h, w2_ref[...],
                          preferred_element_type=jnp.float32) * dv


def _tc3_body(p_ref, x2_ref, dv_ref, b2_ref, out_ref):
    s = p_ref[0] + p_ref[1] - x2_ref[...]
    out_ref[...] = dv_ref[...] * s + b2_ref[...]


def _tc1(deg_t, x_pad, w1):
    return pl.pallas_call(
        _tc1_body,
        grid=(NP // BLK,),
        in_specs=[
            pl.BlockSpec((BLK, NC), lambda i: (i, 0)),
            pl.BlockSpec((BLK, D_IN), lambda i: (i, 0)),
            pl.BlockSpec((D_IN, D_HID), lambda i: (0, 0)),
        ],
        out_specs=[
            pl.BlockSpec((BLK, D_HID), lambda i: (i, 0)),
            pl.BlockSpec((BLK, 1), lambda i: (i, 0)),
        ],
        out_shape=[
            jax.ShapeDtypeStruct((NP, D_HID), jnp.float32),
            jax.ShapeDtypeStruct((NP, 1), jnp.float32),
        ],
    )(deg_t, x_pad, w1)


def _tc2(p1, x1, dv, b1r, w2p):
    return pl.pallas_call(
        _tc2_body,
        grid=(NP // BLK,),
        in_specs=[
            pl.BlockSpec((NC, BLK, D_HID), lambda i: (0, i, 0)),
            pl.BlockSpec((BLK, D_HID), lambda i: (i, 0)),
            pl.BlockSpec((BLK, 1), lambda i: (i, 0)),
            pl.BlockSpec((1, D_HID), lambda i: (0, 0)),
            pl.BlockSpec((D_HID, D_OUT_PAD), lambda i: (0, 0)),
        ],
        out_specs=pl.BlockSpec((BLK, D_OUT_PAD), lambda i: (i, 0)),
        out_shape=jax.ShapeDtypeStruct((NP, D_OUT_PAD), jnp.float32),
    )(p1, x1, dv, b1r, w2p)


def _tc3(p2, x2, dv, b2r):
    return pl.pallas_call(
        _tc3_body,
        grid=(NP // BLK,),
        in_specs=[
            pl.BlockSpec((NC, BLK, D_OUT_PAD), lambda i: (0, i, 0)),
            pl.BlockSpec((BLK, D_OUT_PAD), lambda i: (i, 0)),
            pl.BlockSpec((BLK, 1), lambda i: (i, 0)),
            pl.BlockSpec((1, D_OUT_PAD), lambda i: (0, 0)),
        ],
        out_specs=pl.BlockSpec((BLK, D_OUT_PAD), lambda i: (i, 0)),
        out_shape=jax.ShapeDtypeStruct((NP, D_OUT_PAD), jnp.float32),
    )(p2, x2, dv, b2r)


def kernel(x, edge_index, W1, b1, W2, b2):
    ei = edge_index.astype(jnp.int32)
    row3 = ei[0].reshape(NW, CPT, CH)
    col3 = ei[1].reshape(NW, CPT, CH)
    ones = jnp.ones((NP,), jnp.float32)
    x_pad = jnp.pad(x, ((0, NP - N_NODES), (0, 0)))
    b1r = b1.reshape(1, D_HID)
    w2p = jnp.pad(W2, ((0, 0), (0, D_OUT_PAD - D_OUT)))
    b2r = jnp.pad(b2, (0, D_OUT_PAD - D_OUT)).reshape(1, D_OUT_PAD)

    deg = _deg_kernel(col3, ones)
    x1, dv = _tc1(deg.T, x_pad, W1)
    p1 = _agg1(x1, row3, col3)
    x2 = _tc2(p1, x1, dv, b1r, w2p)
    p2 = _agg2(x2, row3, col3)
    out = _tc3(p2, x2, dv, b2r)
    return out[:N_NODES, :D_OUT]


# speedup vs baseline: 19.7287x; 19.7287x over previous
import functools

import jax
import jax.numpy as jnp
from jax import lax
from jax.experimental import pallas as pl
from jax.experimental.pallas import tpu as pltpu
from jax.experimental.pallas import tpu_sc as plsc

N_NODES = 10000
N_EDGES = 320000
D_IN = 128
D_HID = 128
D_OUT = 40
DP = 128

NC = 2
NS = 16
NW = NC * NS
EPW = N_EDGES // NW
CH = 80
CPT = EPW // CH
VL = 16

NP = 10240
RPT = NP // NS

BLK = 1024

_mesh = plsc.VectorSubcoreMesh(core_axis_name="c", subcore_axis_name="s")


def _wid():
    return lax.axis_index("c") * NS + lax.axis_index("s")


@functools.partial(
    pl.kernel,
    out_type=jax.ShapeDtypeStruct((NW * NP,), jnp.float32),
    mesh=_mesh,
    scratch_types=[
        pltpu.VMEM((EPW,), jnp.int32),
        pltpu.VMEM((NP,), jnp.float32),
    ],
    compiler_params=pltpu.CompilerParams(needs_layout_passes=False),
)
def _deg_kernel(col_hbm, zeros_hbm, out_hbm, colv, hist):
    wid = _wid()
    pltpu.sync_copy(col_hbm.at[wid], colv)
    pltpu.sync_copy(zeros_hbm, hist)

    def body(j, carry):
        idx = colv[pl.ds(j * VL, VL)]
        cnt, last = plsc.scan_count(idx)
        plsc.addupdate_scatter(hist, [idx], cnt.astype(jnp.float32),
                               mask=last)
        return carry

    lax.fori_loop(0, EPW // VL, body, 0, unroll=False)
    pltpu.sync_copy(hist, out_hbm.at[pl.ds(wid * NP, NP)])


@functools.partial(
    pl.kernel,
    out_type=jax.ShapeDtypeStruct((NC, NP, DP), jnp.float32),
    mesh=_mesh,
    scratch_types=[
        pltpu.VMEM((CPT, CH), jnp.int32),
        pltpu.VMEM((CPT, CH), jnp.int32),
        pltpu.VMEM((CH, DP), jnp.float32),
        pltpu.VMEM_SHARED((NP, DP), jnp.float32),
    ],
)
def _agg(x_hbm, row_hbm, col_hbm, out_hbm, rowv, colv, buf, acc):
    cid = lax.axis_index("c")
    sid = lax.axis_index("s")
    pltpu.sync_copy(row_hbm.at[_wid()], rowv)
    pltpu.sync_copy(col_hbm.at[_wid()], colv)
    pltpu.sync_copy(x_hbm.at[pl.ds(sid * RPT, RPT)],
                    acc.at[pl.ds(sid * RPT, RPT)])
    plsc.subcore_barrier()

    def body(j, carry):
        pltpu.sync_copy(x_hbm.at[rowv.at[j]], buf)
        pltpu.sync_copy(buf, acc.at[colv.at[j]], add=True)
        return carry

    lax.fori_loop(0, CPT, body, 0, unroll=False)
    plsc.subcore_barrier()
    pltpu.sync_copy(acc.at[pl.ds(sid * RPT, RPT)],
                    out_hbm.at[cid].at[pl.ds(sid * RPT, RPT)])


def _tc1_body(deg_ref, x_ref, w1_ref, x1_ref, dv_ref):
    deg = jnp.sum(deg_ref[...], axis=1, keepdims=True) + 1.0
    dv = lax.rsqrt(deg)
    dv_ref[...] = dv
    x1_ref[...] = jnp.dot(x_ref[...] * dv, w1_ref[...],
                          preferred_element_type=jnp.float32)


def _tc2_body(p_ref, x1_ref, dv_ref, b1_ref, w2_ref, x2_ref):
    x1 = x1_ref[...]
    s = p_ref[0] + p_ref[1] - x1
    dv = dv_ref[...]
    h = jnp.maximum(dv * s + b1_ref[...], 0.0)
    x2_ref[...] = jnp.dot(h, w2_ref[...],
                          preferred_element_type=jnp.float32) * dv


def _tc3_body(p_ref, x2_ref, dv_ref, b2_ref, out_ref):
    s = p_ref[0] + p_ref[1] - x2_ref[...]
    out_ref[...] = dv_ref[...] * s + b2_ref[...]


def _tc1(deg_t, x_pad, w1):
    return pl.pallas_call(
        _tc1_body,
        grid=(NP // BLK,),
        in_specs=[
            pl.BlockSpec((BLK, NW), lambda i: (i, 0)),
            pl.BlockSpec((BLK, D_IN), lambda i: (i, 0)),
            pl.BlockSpec((D_IN, D_HID), lambda i: (0, 0)),
        ],
        out_specs=[
            pl.BlockSpec((BLK, D_HID), lambda i: (i, 0)),
            pl.BlockSpec((BLK, 1), lambda i: (i, 0)),
        ],
        out_shape=[
            jax.ShapeDtypeStruct((NP, D_HID), jnp.float32),
            jax.ShapeDtypeStruct((NP, 1), jnp.float32),
        ],
    )(deg_t, x_pad, w1)


def _tc2(p1, x1, dv, b1r, w2p):
    return pl.pallas_call(
        _tc2_body,
        grid=(NP // BLK,),
        in_specs=[
            pl.BlockSpec((NC, BLK, D_HID), lambda i: (0, i, 0)),
            pl.BlockSpec((BLK, D_HID), lambda i: (i, 0)),
            pl.BlockSpec((BLK, 1), lambda i: (i, 0)),
            pl.BlockSpec((1, D_HID), lambda i: (0, 0)),
            pl.BlockSpec((D_HID, DP), lambda i: (0, 0)),
        ],
        out_specs=pl.BlockSpec((BLK, DP), lambda i: (i, 0)),
        out_shape=jax.ShapeDtypeStruct((NP, DP), jnp.float32),
    )(p1, x1, dv, b1r, w2p)


def _tc3(p2, x2, dv, b2r):
    return pl.pallas_call(
        _tc3_body,
        grid=(NP // BLK,),
        in_specs=[
            pl.BlockSpec((NC, BLK, DP), lambda i: (0, i, 0)),
            pl.BlockSpec((BLK, DP), lambda i: (i, 0)),
            pl.BlockSpec((BLK, 1), lambda i: (i, 0)),
            pl.BlockSpec((1, DP), lambda i: (0, 0)),
        ],
        out_specs=pl.BlockSpec((BLK, DP), lambda i: (i, 0)),
        out_shape=jax.ShapeDtypeStruct((NP, DP), jnp.float32),
    )(p2, x2, dv, b2r)


def kernel(x, edge_index, W1, b1, W2, b2):
    ei = edge_index.astype(jnp.int32)
    row3 = ei[0].reshape(NW, CPT, CH)
    col3 = ei[1].reshape(NW, CPT, CH)
    col2 = ei[1].reshape(NW, EPW)
    zeros = jnp.zeros((NP,), jnp.float32)
    x_pad = jnp.pad(x, ((0, NP - N_NODES), (0, 0)))
    b1r = b1.reshape(1, D_HID)
    w2p = jnp.pad(W2, ((0, 0), (0, DP - D_OUT)))
    b2r = jnp.pad(b2, (0, DP - D_OUT)).reshape(1, DP)

    deg = _deg_kernel(col2, zeros)
    deg_t = deg.reshape(NW, NP).T
    x1, dv = _tc1(deg_t, x_pad, W1)
    p1 = _agg(x1, row3, col3)
    x2 = _tc2(p1, x1, dv, b1r, w2p)
    p2 = _agg(x2, row3, col3)
    out = _tc3(p2, x2, dv, b2r)
    return out[:N_NODES, :D_OUT]


# speedup vs baseline: 24.7245x; 1.2532x over previous
import functools

import jax
import jax.numpy as jnp
from jax import lax
from jax.experimental import pallas as pl
from jax.experimental.pallas import tpu as pltpu
from jax.experimental.pallas import tpu_sc as plsc

N_NODES = 10000
N_EDGES = 320000
D_IN = 128
D_HID = 128
D_OUT = 40
DP = 128
DP2 = 64

NC = 2
NS = 16
NW = NC * NS
CH = 128
CPT = 80
EPW = CPT * CH
NE_PAD = NW * EPW
VL = 16
NB = 4
PF = 2

NP = 10240
RPT = NP // NS

BLK = 1024

_mesh = plsc.VectorSubcoreMesh(core_axis_name="c", subcore_axis_name="s")


def _wid():
    return lax.axis_index("c") * NS + lax.axis_index("s")


@functools.partial(
    pl.kernel,
    out_type=jax.ShapeDtypeStruct((NW * NP,), jnp.float32),
    mesh=_mesh,
    scratch_types=[
        pltpu.VMEM((EPW,), jnp.int32),
        pltpu.VMEM((NP,), jnp.float32),
    ],
    compiler_params=pltpu.CompilerParams(needs_layout_passes=False),
)
def _deg_kernel(col_hbm, zeros_hbm, out_hbm, colv, hist):
    wid = _wid()
    pltpu.sync_copy(col_hbm.at[wid], colv)
    pltpu.sync_copy(zeros_hbm, hist)

    def body(j, carry):
        idx = colv[pl.ds(j * VL, VL)]
        cnt, last = plsc.scan_count(idx)
        plsc.addupdate_scatter(hist, [idx], cnt.astype(jnp.float32),
                               mask=last)
        return carry

    lax.fori_loop(0, EPW // VL, body, 0, unroll=False)
    pltpu.sync_copy(hist, out_hbm.at[pl.ds(wid * NP, NP)])


def _make_agg(dp, tc_tiling):
    params = None
    if not tc_tiling:
        params = pltpu.CompilerParams(use_tc_tiling_on_sc=False)

    @functools.partial(
        pl.kernel,
        out_type=jax.ShapeDtypeStruct((NC, NP, dp), jnp.float32),
        mesh=_mesh,
        scratch_types=[
            pltpu.VMEM((CPT, CH), jnp.int32),
            pltpu.VMEM((CPT, CH), jnp.int32),
            pltpu.VMEM((CH, dp), jnp.float32),
            pltpu.VMEM_SHARED((NP, dp), jnp.float32),
        ],
        compiler_params=params,
    )
    def agg(x_hbm, row_hbm, col_hbm, out_hbm, rowv, colv, buf, acc):
        cid = lax.axis_index("c")
        sid = lax.axis_index("s")
        pltpu.sync_copy(row_hbm.at[_wid()], rowv)
        pltpu.sync_copy(col_hbm.at[_wid()], colv)
        pltpu.sync_copy(x_hbm.at[pl.ds(sid * RPT, RPT)],
                        acc.at[pl.ds(sid * RPT, RPT)])
        plsc.subcore_barrier()

        def body(j, carry):
            pltpu.sync_copy(x_hbm.at[rowv.at[j]], buf)
            pltpu.sync_copy(buf, acc.at[colv.at[j]], add=True)
            return carry

        lax.fori_loop(0, CPT, body, 0, unroll=False)
        plsc.subcore_barrier()
        pltpu.sync_copy(acc.at[pl.ds(sid * RPT, RPT)],
                        out_hbm.at[cid].at[pl.ds(sid * RPT, RPT)])

    return agg


_agg1 = _make_agg(DP, False)
_agg2 = _make_agg(DP2, False)


def _tc1_body(deg_ref, x_ref, w1_ref, x1_ref, dv_ref):
    deg = jnp.sum(deg_ref[...], axis=1, keepdims=True) + 1.0
    dv = lax.rsqrt(deg)
    dv_ref[...] = dv
    x1_ref[...] = jnp.dot(x_ref[...] * dv, w1_ref[...],
                          preferred_element_type=jnp.float32)


def _tc2_body(p_ref, x1_ref, dv_ref, b1_ref, w2_ref, x2_ref):
    x1 = x1_ref[...]
    s = p_ref[0] + p_ref[1] - x1
    dv = dv_ref[...]
    h = jnp.maximum(dv * s + b1_ref[...], 0.0)
    x2_ref[...] = jnp.dot(h, w2_ref[...],
                          preferred_element_type=jnp.float32) * dv


def _tc3_body(p_ref, x2_ref, dv_ref, b2_ref, out_ref):
    s = p_ref[0] + p_ref[1] - x2_ref[...]
    out_ref[...] = dv_ref[...] * s + b2_ref[...]


def _tc1(deg_t, x_pad, w1):
    return pl.pallas_call(
        _tc1_body,
        grid=(NP // BLK,),
        in_specs=[
            pl.BlockSpec((BLK, NW), lambda i: (i, 0)),
            pl.BlockSpec((BLK, D_IN), lambda i: (i, 0)),
            pl.BlockSpec((D_IN, D_HID), lambda i: (0, 0)),
        ],
        out_specs=[
            pl.BlockSpec((BLK, D_HID), lambda i: (i, 0)),
            pl.BlockSpec((BLK, 1), lambda i: (i, 0)),
        ],
        out_shape=[
            jax.ShapeDtypeStruct((NP, D_HID), jnp.float32),
            jax.ShapeDtypeStruct((NP, 1), jnp.float32),
        ],
    )(deg_t, x_pad, w1)


def _tc2(p1, x1, dv, b1r, w2p):
    return pl.pallas_call(
        _tc2_body,
        grid=(NP // BLK,),
        in_specs=[
            pl.BlockSpec((NC, BLK, D_HID), lambda i: (0, i, 0)),
            pl.BlockSpec((BLK, D_HID), lambda i: (i, 0)),
            pl.BlockSpec((BLK, 1), lambda i: (i, 0)),
            pl.BlockSpec((1, D_HID), lambda i: (0, 0)),
            pl.BlockSpec((D_HID, DP2), lambda i: (0, 0)),
        ],
        out_specs=pl.BlockSpec((BLK, DP2), lambda i: (i, 0)),
        out_shape=jax.ShapeDtypeStruct((NP, DP2), jnp.float32),
    )(p1, x1, dv, b1r, w2p)


def _tc3(p2, x2, dv, b2r):
    return pl.pallas_call(
        _tc3_body,
        grid=(NP // BLK,),
        in_specs=[
            pl.BlockSpec((NC, BLK, DP2), lambda i: (0, i, 0)),
            pl.BlockSpec((BLK, DP2), lambda i: (i, 0)),
            pl.BlockSpec((BLK, 1), lambda i: (i, 0)),
            pl.BlockSpec((1, DP2), lambda i: (0, 0)),
        ],
        out_specs=pl.BlockSpec((BLK, DP2), lambda i: (i, 0)),
        out_shape=jax.ShapeDtypeStruct((NP, DP2), jnp.float32),
    )(p2, x2, dv, b2r)


def kernel(x, edge_index, W1, b1, W2, b2):
    ei = edge_index.astype(jnp.int32)
    npad = NE_PAD - N_EDGES
    it = lax.iota(jnp.int32, npad)
    rowp = jnp.concatenate([ei[0], it % N_NODES])
    colp = jnp.concatenate([ei[1], N_NODES + it % (NP - N_NODES)])
    row3 = rowp.reshape(NW, CPT, CH)
    col3 = colp.reshape(NW, CPT, CH)
    col2 = colp.reshape(NW, EPW)
    zeros = jnp.zeros((NP,), jnp.float32)
    x_pad = jnp.pad(x, ((0, NP - N_NODES), (0, 0)))
    b1r = b1.reshape(1, D_HID)
    w2p = jnp.pad(W2, ((0, 0), (0, DP2 - D_OUT)))
    b2r = jnp.pad(b2, (0, DP2 - D_OUT)).reshape(1, DP2)

    deg = _deg_kernel(col2, zeros)
    deg_t = deg.reshape(NW, NP).T
    x1, dv = _tc1(deg_t, x_pad, W1)
    p1 = _agg1(x1, row3, col3)
    x2 = _tc2(p1, x1, dv, b1r, w2p)
    p2 = _agg2(x2, row3, col3)
    out = _tc3(p2, x2, dv, b2r)
    return out[:N_NODES, :D_OUT]


# speedup vs baseline: 25.0367x; 1.0126x over previous
import functools

import jax
import jax.numpy as jnp
from jax import lax
from jax.experimental import pallas as pl
from jax.experimental.pallas import tpu as pltpu
from jax.experimental.pallas import tpu_sc as plsc

N_NODES = 10000
N_EDGES = 320000
D_IN = 128
D_HID = 128
D_OUT = 40
DP = 64

NC = 2
NS = 16
NW = NC * NS
CH = 128
CPT = 80
EPW = CPT * CH
NE_PAD = NW * EPW
VL = 16

NP = 10240
RPT = NP // NS

BLK = 1024

_mesh = plsc.VectorSubcoreMesh(core_axis_name="c", subcore_axis_name="s")


def _wid():
    return lax.axis_index("c") * NS + lax.axis_index("s")


@functools.partial(
    pl.kernel,
    out_type=jax.ShapeDtypeStruct((NW * NP,), jnp.float32),
    mesh=_mesh,
    scratch_types=[
        pltpu.VMEM((EPW,), jnp.int32),
        pltpu.VMEM((NP,), jnp.float32),
    ],
    compiler_params=pltpu.CompilerParams(needs_layout_passes=False),
)
def _deg_kernel(col_hbm, zeros_hbm, out_hbm, colv, hist):
    wid = _wid()
    pltpu.sync_copy(col_hbm.at[wid], colv)
    pltpu.sync_copy(zeros_hbm, hist)

    def body(j, carry):
        idx = colv[pl.ds(j * VL, VL)]
        cnt, last = plsc.scan_count(idx)
        plsc.addupdate_scatter(hist, [idx], cnt.astype(jnp.float32),
                               mask=last)
        return carry

    lax.fori_loop(0, EPW // VL, body, 0, unroll=False)
    pltpu.sync_copy(hist, out_hbm.at[pl.ds(wid * NP, NP)])


def _make_agg(h_passes):
    @functools.partial(
        pl.kernel,
        out_type=jax.ShapeDtypeStruct((NC, h_passes, NP, DP), jnp.float32),
        mesh=_mesh,
        scratch_types=[
            pltpu.VMEM((CPT, CH), jnp.int32),
            pltpu.VMEM((CPT, CH), jnp.int32),
            pltpu.VMEM((2, CH, DP), jnp.float32),
            pltpu.VMEM_SHARED((NP, DP), jnp.float32),
            pltpu.SemaphoreType.DMA,
            pltpu.SemaphoreType.DMA,
        ],
        compiler_params=pltpu.CompilerParams(use_tc_tiling_on_sc=False),
    )
    def agg(x_hbm, row_hbm, col_hbm, out_hbm, rowv, colv, buf, acc,
            gsem0, gsem1):
        cid = lax.axis_index("c")
        sid = lax.axis_index("s")
        sems = (gsem0, gsem1)
        pltpu.sync_copy(row_hbm.at[_wid()], rowv)
        pltpu.sync_copy(col_hbm.at[_wid()], colv)

        for h in range(h_passes):
            xh = x_hbm.at[h]
            pltpu.async_copy(xh.at[rowv.at[0]], buf.at[0], gsem0)
            pltpu.sync_copy(xh.at[pl.ds(sid * RPT, RPT)],
                            acc.at[pl.ds(sid * RPT, RPT)])
            plsc.subcore_barrier()

            def outer(j0, carry):
                for b in range(2):
                    j = j0 * 2 + b
                    pltpu.make_async_copy(xh.at[rowv.at[j]], buf.at[b],
                                          sems[b]).wait()

                    @pl.when(j + 1 < CPT)
                    def _():
                        pltpu.async_copy(xh.at[rowv.at[j + 1]],
                                         buf.at[1 - b], sems[1 - b])

                    pltpu.sync_copy(buf.at[b], acc.at[colv.at[j]],
                                    add=True)
                return carry

            lax.fori_loop(0, CPT // 2, outer, 0, unroll=False)
            plsc.subcore_barrier()
            pltpu.sync_copy(acc.at[pl.ds(sid * RPT, RPT)],
                            out_hbm.at[cid].at[h].at[pl.ds(sid * RPT, RPT)])

    return agg


_agg1 = _make_agg(2)
_agg2 = _make_agg(1)


def _tc1_body(deg_ref, x_ref, w1_ref, x1_ref, dv_ref):
    deg = jnp.sum(deg_ref[...], axis=1, keepdims=True) + 1.0
    dv = lax.rsqrt(deg)
    dv_ref[...] = dv
    x1 = jnp.dot(x_ref[...] * dv, w1_ref[...],
                 preferred_element_type=jnp.float32)
    x1_ref[0] = x1[:, :DP]
    x1_ref[1] = x1[:, DP:]


def _tc2_body(p_ref, x1_ref, dv_ref, b1_ref, w2_ref, x2_ref):
    s0 = p_ref[0, 0] + p_ref[1, 0] - x1_ref[0]
    s1 = p_ref[0, 1] + p_ref[1, 1] - x1_ref[1]
    s = jnp.concatenate([s0, s1], axis=1)
    dv = dv_ref[...]
    h = jnp.maximum(dv * s + b1_ref[...], 0.0)
    x2_ref[0] = jnp.dot(h, w2_ref[...],
                        preferred_element_type=jnp.float32) * dv


def _tc3_body(p_ref, x2_ref, dv_ref, b2_ref, out_ref):
    s = p_ref[0, 0] + p_ref[1, 0] - x2_ref[0]
    out_ref[...] = dv_ref[...] * s + b2_ref[...]


def _tc1(deg_t, x_pad, w1):
    return pl.pallas_call(
        _tc1_body,
        grid=(NP // BLK,),
        in_specs=[
            pl.BlockSpec((BLK, NW), lambda i: (i, 0)),
            pl.BlockSpec((BLK, D_IN), lambda i: (i, 0)),
            pl.BlockSpec((D_IN, D_HID), lambda i: (0, 0)),
        ],
        out_specs=[
            pl.BlockSpec((2, BLK, DP), lambda i: (0, i, 0)),
            pl.BlockSpec((BLK, 1), lambda i: (i, 0)),
        ],
        out_shape=[
            jax.ShapeDtypeStruct((2, NP, DP), jnp.float32),
            jax.ShapeDtypeStruct((NP, 1), jnp.float32),
        ],
    )(deg_t, x_pad, w1)


def _tc2(p1, x1, dv, b1r, w2p):
    return pl.pallas_call(
        _tc2_body,
        grid=(NP // BLK,),
        in_specs=[
            pl.BlockSpec((NC, 2, BLK, DP), lambda i: (0, 0, i, 0)),
            pl.BlockSpec((2, BLK, DP), lambda i: (0, i, 0)),
            pl.BlockSpec((BLK, 1), lambda i: (i, 0)),
            pl.BlockSpec((1, D_HID), lambda i: (0, 0)),
            pl.BlockSpec((D_HID, DP), lambda i: (0, 0)),
        ],
        out_specs=pl.BlockSpec((1, BLK, DP), lambda i: (0, i, 0)),
        out_shape=jax.ShapeDtypeStruct((1, NP, DP), jnp.float32),
    )(p1, x1, dv, b1r, w2p)


def _tc3(p2, x2, dv, b2r):
    return pl.pallas_call(
        _tc3_body,
        grid=(NP // BLK,),
        in_specs=[
            pl.BlockSpec((NC, 1, BLK, DP), lambda i: (0, 0, i, 0)),
            pl.BlockSpec((1, BLK, DP), lambda i: (0, i, 0)),
            pl.BlockSpec((BLK, 1), lambda i: (i, 0)),
            pl.BlockSpec((1, DP), lambda i: (0, 0)),
        ],
        out_specs=pl.BlockSpec((BLK, DP), lambda i: (i, 0)),
        out_shape=jax.ShapeDtypeStruct((NP, DP), jnp.float32),
    )(p2, x2, dv, b2r)


def kernel(x, edge_index, W1, b1, W2, b2):
    ei = edge_index.astype(jnp.int32)
    npad = NE_PAD - N_EDGES
    it = lax.iota(jnp.int32, npad)
    rowp = jnp.concatenate([ei[0], it % N_NODES])
    colp = jnp.concatenate([ei[1], N_NODES + it % (NP - N_NODES)])
    row3 = rowp.reshape(NW, CPT, CH)
    col3 = colp.reshape(NW, CPT, CH)
    col2 = colp.reshape(NW, EPW)
    zeros = jnp.zeros((NP,), jnp.float32)
    x_pad = jnp.pad(x, ((0, NP - N_NODES), (0, 0)))
    b1r = b1.reshape(1, D_HID)
    w2p = jnp.pad(W2, ((0, 0), (0, DP - D_OUT)))
    b2r = jnp.pad(b2, (0, DP - D_OUT)).reshape(1, DP)

    deg = _deg_kernel(col2, zeros)
    deg_t = deg.reshape(NW, NP).T
    x1, dv = _tc1(deg_t, x_pad, W1)
    p1 = _agg1(x1, row3, col3)
    x2 = _tc2(p1, x1, dv, b1r, w2p)
    p2 = _agg2(x2, row3, col3)
    out = _tc3(p2, x2, dv, b2r)
    return out[:N_NODES, :D_OUT]


# speedup vs baseline: 30.9056x; 1.2344x over previous
import functools

import jax
import jax.numpy as jnp
from jax import lax
from jax.experimental import pallas as pl
from jax.experimental.pallas import tpu as pltpu
from jax.experimental.pallas import tpu_sc as plsc

N_NODES = 10000
N_EDGES = 320000
D_IN = 128
D_HID = 128
D_OUT = 40
DP = 64

NC = 2
NS = 16
NW = NC * NS
CH = 128
CPT = 80
EPW = CPT * CH
NE_PAD = NW * EPW
VL = 16

NP = 10240
RPT = NP // NS

BLK = 1024

_mesh = plsc.VectorSubcoreMesh(core_axis_name="c", subcore_axis_name="s")


def _wid():
    return lax.axis_index("c") * NS + lax.axis_index("s")


@functools.partial(
    pl.kernel,
    out_type=jax.ShapeDtypeStruct((NW * NP,), jnp.float32),
    mesh=_mesh,
    scratch_types=[
        pltpu.VMEM((EPW,), jnp.int32),
        pltpu.VMEM((NP,), jnp.float32),
    ],
    compiler_params=pltpu.CompilerParams(needs_layout_passes=False),
)
def _deg_kernel(col_hbm, zeros_hbm, out_hbm, colv, hist):
    wid = _wid()
    pltpu.sync_copy(col_hbm.at[wid], colv)
    pltpu.sync_copy(zeros_hbm, hist)

    def body(j, carry):
        idx = colv[pl.ds(j * VL, VL)]
        cnt, last = plsc.scan_count(idx)
        plsc.addupdate_scatter(hist, [idx], cnt.astype(jnp.float32),
                               mask=last)
        return carry

    lax.fori_loop(0, EPW // VL, body, 0, unroll=False)
    pltpu.sync_copy(hist, out_hbm.at[pl.ds(wid * NP, NP)])


def _make_agg(h_passes):
    @functools.partial(
        pl.kernel,
        out_type=jax.ShapeDtypeStruct((NC, h_passes, NP, DP), jnp.float32),
        mesh=_mesh,
        scratch_types=[
            pltpu.VMEM((CPT, CH), jnp.int32),
            pltpu.VMEM((CPT, CH), jnp.int32),
            pltpu.VMEM((4, CH, DP), jnp.float32),
            pltpu.VMEM_SHARED((NP, DP), jnp.float32),
            pltpu.SemaphoreType.DMA((4,)),
            pltpu.SemaphoreType.DMA((4,)),
        ],
        compiler_params=pltpu.CompilerParams(use_tc_tiling_on_sc=False),
    )
    def agg(x_hbm, row_hbm, col_hbm, out_hbm, rowv, colv, buf, acc,
            gsem, ssem):
        cid = lax.axis_index("c")
        sid = lax.axis_index("s")
        pltpu.sync_copy(row_hbm.at[_wid()], rowv)
        pltpu.sync_copy(col_hbm.at[_wid()], colv)

        for h in range(h_passes):
            xh = x_hbm.at[h]
            for b in range(2):
                pltpu.async_copy(xh.at[rowv.at[b]], buf.at[b], gsem.at[b])
            pltpu.sync_copy(xh.at[pl.ds(sid * RPT, RPT)],
                            acc.at[pl.ds(sid * RPT, RPT)])
            plsc.subcore_barrier()

            def outer(j0, carry):
                for b in range(4):
                    j = j0 * 4 + b
                    pltpu.make_async_copy(xh.at[rowv.at[j]], buf.at[b],
                                          gsem.at[b]).wait()
                    pltpu.async_copy(buf.at[b], acc.at[colv.at[j]],
                                     ssem.at[b], add=True)
                    p = (b + 2) % 4
                    jp = j + 2

                    @pl.when(jp < CPT)
                    def _():
                        @pl.when(jp >= 4)
                        def _():
                            pltpu.make_async_copy(
                                buf.at[p], acc.at[colv.at[jp - 4]],
                                ssem.at[p]).wait()
                        pltpu.async_copy(xh.at[rowv.at[jp]], buf.at[p],
                                         gsem.at[p])
                return carry

            lax.fori_loop(0, CPT // 4, outer, 0, unroll=False)
            for b in range(4):
                pltpu.make_async_copy(buf.at[b],
                                      acc.at[colv.at[CPT - 4 + b]],
                                      ssem.at[b]).wait()
            plsc.subcore_barrier()
            pltpu.sync_copy(acc.at[pl.ds(sid * RPT, RPT)],
                            out_hbm.at[cid].at[h].at[pl.ds(sid * RPT, RPT)])

    return agg


_agg1 = _make_agg(2)
_agg2 = _make_agg(1)


def _tc1_body(deg_ref, x_ref, w1_ref, x1_ref, dv_ref):
    deg = jnp.sum(deg_ref[...], axis=1, keepdims=True) + 1.0
    dv = lax.rsqrt(deg)
    dv_ref[...] = dv
    x1 = jnp.dot(x_ref[...] * dv, w1_ref[...],
                 preferred_element_type=jnp.float32)
    x1_ref[0] = x1[:, :DP]
    x1_ref[1] = x1[:, DP:]


def _tc2_body(p_ref, x1_ref, dv_ref, b1_ref, w2_ref, x2_ref):
    s0 = p_ref[0, 0] + p_ref[1, 0] - x1_ref[0]
    s1 = p_ref[0, 1] + p_ref[1, 1] - x1_ref[1]
    s = jnp.concatenate([s0, s1], axis=1)
    dv = dv_ref[...]
    h = jnp.maximum(dv * s + b1_ref[...], 0.0)
    x2_ref[0] = jnp.dot(h, w2_ref[...],
                        preferred_element_type=jnp.float32) * dv


def _tc3_body(p_ref, x2_ref, dv_ref, b2_ref, out_ref):
    s = p_ref[0, 0] + p_ref[1, 0] - x2_ref[0]
    out_ref[...] = dv_ref[...] * s + b2_ref[...]


def _tc1(deg_t, x_pad, w1):
    return pl.pallas_call(
        _tc1_body,
        grid=(NP // BLK,),
        in_specs=[
            pl.BlockSpec((BLK, NW), lambda i: (i, 0)),
            pl.BlockSpec((BLK, D_IN), lambda i: (i, 0)),
            pl.BlockSpec((D_IN, D_HID), lambda i: (0, 0)),
        ],
        out_specs=[
            pl.BlockSpec((2, BLK, DP), lambda i: (0, i, 0)),
            pl.BlockSpec((BLK, 1), lambda i: (i, 0)),
        ],
        out_shape=[
            jax.ShapeDtypeStruct((2, NP, DP), jnp.float32),
            jax.ShapeDtypeStruct((NP, 1), jnp.float32),
        ],
    )(deg_t, x_pad, w1)


def _tc2(p1, x1, dv, b1r, w2p):
    return pl.pallas_call(
        _tc2_body,
        grid=(NP // BLK,),
        in_specs=[
            pl.BlockSpec((NC, 2, BLK, DP), lambda i: (0, 0, i, 0)),
            pl.BlockSpec((2, BLK, DP), lambda i: (0, i, 0)),
            pl.BlockSpec((BLK, 1), lambda i: (i, 0)),
            pl.BlockSpec((1, D_HID), lambda i: (0, 0)),
            pl.BlockSpec((D_HID, DP), lambda i: (0, 0)),
        ],
        out_specs=pl.BlockSpec((1, BLK, DP), lambda i: (0, i, 0)),
        out_shape=jax.ShapeDtypeStruct((1, NP, DP), jnp.float32),
    )(p1, x1, dv, b1r, w2p)


def _tc3(p2, x2, dv, b2r):
    return pl.pallas_call(
        _tc3_body,
        grid=(NP // BLK,),
        in_specs=[
            pl.BlockSpec((NC, 1, BLK, DP), lambda i: (0, 0, i, 0)),
            pl.BlockSpec((1, BLK, DP), lambda i: (0, i, 0)),
            pl.BlockSpec((BLK, 1), lambda i: (i, 0)),
            pl.BlockSpec((1, DP), lambda i: (0, 0)),
        ],
        out_specs=pl.BlockSpec((BLK, DP), lambda i: (i, 0)),
        out_shape=jax.ShapeDtypeStruct((NP, DP), jnp.float32),
    )(p2, x2, dv, b2r)


def kernel(x, edge_index, W1, b1, W2, b2):
    ei = edge_index.astype(jnp.int32)
    npad = NE_PAD - N_EDGES
    it = lax.iota(jnp.int32, npad)
    rowp = jnp.concatenate([ei[0], it % N_NODES])
    colp = jnp.concatenate([ei[1], N_NODES + it % (NP - N_NODES)])
    row3 = rowp.reshape(NW, CPT, CH)
    col3 = colp.reshape(NW, CPT, CH)
    col2 = colp.reshape(NW, EPW)
    zeros = jnp.zeros((NP,), jnp.float32)
    x_pad = jnp.pad(x, ((0, NP - N_NODES), (0, 0)))
    b1r = b1.reshape(1, D_HID)
    w2p = jnp.pad(W2, ((0, 0), (0, DP - D_OUT)))
    b2r = jnp.pad(b2, (0, DP - D_OUT)).reshape(1, DP)

    deg = _deg_kernel(col2, zeros)
    deg_t = deg.reshape(NW, NP).T
    x1, dv = _tc1(deg_t, x_pad, W1)
    p1 = _agg1(x1, row3, col3)
    x2 = _tc2(p1, x1, dv, b1r, w2p)
    p2 = _agg2(x2, row3, col3)
    out = _tc3(p2, x2, dv, b2r)
    return out[:N_NODES, :D_OUT]


# speedup vs baseline: 31.2756x; 1.0120x over previous
import functools

import jax
import jax.numpy as jnp
from jax import lax
from jax.experimental import pallas as pl
from jax.experimental.pallas import tpu as pltpu
from jax.experimental.pallas import tpu_sc as plsc

N_NODES = 10000
N_EDGES = 320000
D_IN = 128
D_HID = 128
D_OUT = 40
DP = 64

NC = 2
NS = 16
NW = NC * NS
CH = 128
CPT = 80
EPW = CPT * CH
NE_PAD = NW * EPW
VL = 16

NP = 10240
RPT = NP // NS

BLK = 1024

_mesh = plsc.VectorSubcoreMesh(core_axis_name="c", subcore_axis_name="s")


def _wid():
    return lax.axis_index("c") * NS + lax.axis_index("s")


@functools.partial(
    pl.kernel,
    out_type=jax.ShapeDtypeStruct((NW * NP,), jnp.float32),
    mesh=_mesh,
    scratch_types=[
        pltpu.VMEM((EPW,), jnp.int32),
        pltpu.VMEM((NP,), jnp.float32),
    ],
    compiler_params=pltpu.CompilerParams(needs_layout_passes=False),
)
def _deg_kernel(col_hbm, zeros_hbm, out_hbm, colv, hist):
    wid = _wid()
    pltpu.sync_copy(col_hbm.at[wid], colv)
    pltpu.sync_copy(zeros_hbm, hist)

    def body(j, carry):
        idx = colv[pl.ds(j * VL, VL)]
        cnt, last = plsc.scan_count(idx)
        plsc.addupdate_scatter(hist, [idx], cnt.astype(jnp.float32),
                               mask=last)
        return carry

    lax.fori_loop(0, EPW // VL, body, 0, unroll=False)
    pltpu.sync_copy(hist, out_hbm.at[pl.ds(wid * NP, NP)])


def _make_agg(h_passes):
    @functools.partial(
        pl.kernel,
        out_type=jax.ShapeDtypeStruct((NC, h_passes, NP, DP), jnp.float32),
        mesh=_mesh,
        scratch_types=[
            pltpu.VMEM((CPT, CH), jnp.int32),
            pltpu.VMEM((CPT, CH), jnp.int32),
            pltpu.VMEM((5, CH, DP), jnp.float32),
            pltpu.VMEM_SHARED((NP, DP), jnp.float32),
            pltpu.SemaphoreType.DMA((5,)),
            pltpu.SemaphoreType.DMA((5,)),
        ],
        compiler_params=pltpu.CompilerParams(use_tc_tiling_on_sc=False),
    )
    def agg(x_hbm, row_hbm, col_hbm, out_hbm, rowv, colv, buf, acc,
            gsem, ssem):
        cid = lax.axis_index("c")
        sid = lax.axis_index("s")
        pltpu.sync_copy(row_hbm.at[_wid()], rowv)
        pltpu.sync_copy(col_hbm.at[_wid()], colv)

        for h in range(h_passes):
            xh = x_hbm.at[h]
            for b in range(2):
                pltpu.async_copy(xh.at[rowv.at[b]], buf.at[b], gsem.at[b])
            pltpu.sync_copy(xh.at[pl.ds(sid * RPT, RPT)],
                            acc.at[pl.ds(sid * RPT, RPT)])
            plsc.subcore_barrier()

            def outer(j0, carry):
                for b in range(5):
                    j = j0 * 5 + b
                    pltpu.make_async_copy(xh.at[rowv.at[j]], buf.at[b],
                                          gsem.at[b]).wait()
                    pltpu.async_copy(buf.at[b], acc.at[colv.at[j]],
                                     ssem.at[b], add=True)
                    p = (b + 2) % 5
                    jp = j + 2

                    @pl.when(jp < CPT)
                    def _():
                        @pl.when(jp >= 5)
                        def _():
                            pltpu.make_async_copy(
                                buf.at[p], acc.at[colv.at[jp - 5]],
                                ssem.at[p]).wait()
                        pltpu.async_copy(xh.at[rowv.at[jp]], buf.at[p],
                                         gsem.at[p])
                return carry

            lax.fori_loop(0, CPT // 5, outer, 0, unroll=False)
            for b in range(5):
                pltpu.make_async_copy(buf.at[b],
                                      acc.at[colv.at[CPT - 5 + b]],
                                      ssem.at[b]).wait()
            plsc.subcore_barrier()
            pltpu.sync_copy(acc.at[pl.ds(sid * RPT, RPT)],
                            out_hbm.at[cid].at[h].at[pl.ds(sid * RPT, RPT)])

    return agg


_agg1 = _make_agg(2)
_agg2 = _make_agg(1)


def _tc1_body(deg_ref, x_ref, w1_ref, x1_ref, dv_ref):
    deg = jnp.sum(deg_ref[...], axis=0)[:, None] + 1.0
    dv = lax.rsqrt(deg)
    dv_ref[...] = dv
    x1 = jnp.dot(x_ref[...] * dv, w1_ref[...],
                 preferred_element_type=jnp.float32)
    x1_ref[0] = x1[:, :DP]
    x1_ref[1] = x1[:, DP:]


def _tc2_body(p_ref, x1_ref, dv_ref, b1_ref, w2_ref, x2_ref):
    s0 = p_ref[0, 0] + p_ref[1, 0] - x1_ref[0]
    s1 = p_ref[0, 1] + p_ref[1, 1] - x1_ref[1]
    s = jnp.concatenate([s0, s1], axis=1)
    dv = dv_ref[...]
    h = jnp.maximum(dv * s + b1_ref[...], 0.0)
    x2_ref[0] = jnp.dot(h, w2_ref[...],
                        preferred_element_type=jnp.float32) * dv


def _tc3_body(p_ref, x2_ref, dv_ref, b2_ref, out_ref):
    s = p_ref[0, 0] + p_ref[1, 0] - x2_ref[0]
    out_ref[...] = dv_ref[...] * s + b2_ref[...]


def _tc1(deg_t, x_pad, w1):
    return pl.pallas_call(
        _tc1_body,
        grid=(NP // BLK,),
        in_specs=[
            pl.BlockSpec((NW, BLK), lambda i: (0, i)),
            pl.BlockSpec((BLK, D_IN), lambda i: (i, 0)),
            pl.BlockSpec((D_IN, D_HID), lambda i: (0, 0)),
        ],
        out_specs=[
            pl.BlockSpec((2, BLK, DP), lambda i: (0, i, 0)),
            pl.BlockSpec((BLK, 1), lambda i: (i, 0)),
        ],
        out_shape=[
            jax.ShapeDtypeStruct((2, NP, DP), jnp.float32),
            jax.ShapeDtypeStruct((NP, 1), jnp.float32),
        ],
    )(deg_t, x_pad, w1)


def _tc2(p1, x1, dv, b1r, w2p):
    return pl.pallas_call(
        _tc2_body,
        grid=(NP // BLK,),
        in_specs=[
            pl.BlockSpec((NC, 2, BLK, DP), lambda i: (0, 0, i, 0)),
            pl.BlockSpec((2, BLK, DP), lambda i: (0, i, 0)),
            pl.BlockSpec((BLK, 1), lambda i: (i, 0)),
            pl.BlockSpec((1, D_HID), lambda i: (0, 0)),
            pl.BlockSpec((D_HID, DP), lambda i: (0, 0)),
        ],
        out_specs=pl.BlockSpec((1, BLK, DP), lambda i: (0, i, 0)),
        out_shape=jax.ShapeDtypeStruct((1, NP, DP), jnp.float32),
    )(p1, x1, dv, b1r, w2p)


def _tc3(p2, x2, dv, b2r):
    return pl.pallas_call(
        _tc3_body,
        grid=(NP // BLK,),
        in_specs=[
            pl.BlockSpec((NC, 1, BLK, DP), lambda i: (0, 0, i, 0)),
            pl.BlockSpec((1, BLK, DP), lambda i: (0, i, 0)),
            pl.BlockSpec((BLK, 1), lambda i: (i, 0)),
            pl.BlockSpec((1, DP), lambda i: (0, 0)),
        ],
        out_specs=pl.BlockSpec((BLK, DP), lambda i: (i, 0)),
        out_shape=jax.ShapeDtypeStruct((NP, DP), jnp.float32),
    )(p2, x2, dv, b2r)


def kernel(x, edge_index, W1, b1, W2, b2):
    ei = edge_index.astype(jnp.int32)
    npad = NE_PAD - N_EDGES
    it = lax.iota(jnp.int32, npad)
    rowp = jnp.concatenate([ei[0], it % N_NODES])
    colp = jnp.concatenate([ei[1], N_NODES + it % (NP - N_NODES)])
    row3 = rowp.reshape(NW, CPT, CH)
    col3 = colp.reshape(NW, CPT, CH)
    col2 = colp.reshape(NW, EPW)
    zeros = jnp.zeros((NP,), jnp.float32)
    x_pad = jnp.pad(x, ((0, NP - N_NODES), (0, 0)))
    b1r = b1.reshape(1, D_HID)
    w2p = jnp.pad(W2, ((0, 0), (0, DP - D_OUT)))
    b2r = jnp.pad(b2, (0, DP - D_OUT)).reshape(1, DP)

    deg = _deg_kernel(col2, zeros)
    x1, dv = _tc1(deg.reshape(NW, NP), x_pad, W1)
    p1 = _agg1(x1, row3, col3)
    x2 = _tc2(p1, x1, dv, b1r, w2p)
    p2 = _agg2(x2, row3, col3)
    out = _tc3(p2, x2, dv, b2r)
    return out[:N_NODES, :D_OUT]


# speedup vs baseline: 34.3227x; 1.0974x over previous
import functools

import jax
import jax.numpy as jnp
from jax import lax
from jax.experimental import pallas as pl
from jax.experimental.pallas import tpu as pltpu
from jax.experimental.pallas import tpu_sc as plsc

N_NODES = 10000
N_EDGES = 320000
D_IN = 128
D_HID = 128
D_OUT = 40
DP = 64

NC = 2
NS = 16
NW = NC * NS
CH = 128
CPT = 80
EPW = CPT * CH
NE_PAD = NW * EPW
VL = 16

NP = 10240
RPT = NP // NS

BLK = 1024

_mesh = plsc.VectorSubcoreMesh(core_axis_name="c", subcore_axis_name="s")


def _wid():
    return lax.axis_index("c") * NS + lax.axis_index("s")


@functools.partial(
    pl.kernel,
    out_type=jax.ShapeDtypeStruct((NW * NP,), jnp.float32),
    mesh=_mesh,
    scratch_types=[
        pltpu.VMEM((EPW,), jnp.int32),
        pltpu.VMEM((NP,), jnp.float32),
    ],
    compiler_params=pltpu.CompilerParams(needs_layout_passes=False),
)
def _deg_kernel(col_hbm, zeros_hbm, out_hbm, colv, hist):
    wid = _wid()
    pltpu.sync_copy(col_hbm.at[wid], colv)
    pltpu.sync_copy(zeros_hbm, hist)

    def body(j, carry):
        idx = colv[pl.ds(j * VL, VL)]
        cnt, last = plsc.scan_count(idx)
        plsc.addupdate_scatter(hist, [idx], cnt.astype(jnp.float32),
                               mask=last)
        return carry

    lax.fori_loop(0, EPW // VL, body, 0, unroll=False)
    pltpu.sync_copy(hist, out_hbm.at[pl.ds(wid * NP, NP)])


def _make_agg(h_passes):
    @functools.partial(
        pl.kernel,
        out_type=jax.ShapeDtypeStruct((NC, h_passes, NP, DP), jnp.float32),
        mesh=_mesh,
        scratch_types=[
            pltpu.VMEM((CPT, CH), jnp.int32),
            pltpu.VMEM((CPT, CH), jnp.int32),
            pltpu.VMEM((5, CH, DP), jnp.float32),
            pltpu.VMEM_SHARED((NP, DP), jnp.float32),
            pltpu.SemaphoreType.DMA((5,)),
            pltpu.SemaphoreType.DMA((5,)),
        ],
        compiler_params=pltpu.CompilerParams(use_tc_tiling_on_sc=False),
    )
    def agg(x_hbm, row_hbm, col_hbm, out_hbm, rowv, colv, buf, acc,
            gsem, ssem):
        cid = lax.axis_index("c")
        sid = lax.axis_index("s")
        pltpu.sync_copy(row_hbm.at[_wid()], rowv)
        pltpu.sync_copy(col_hbm.at[_wid()], colv)

        for h in range(h_passes):
            xh = x_hbm.at[h]
            for b in range(3):
                pltpu.async_copy(xh.at[rowv.at[b]], buf.at[b], gsem.at[b])
            pltpu.sync_copy(xh.at[pl.ds(sid * RPT, RPT)],
                            acc.at[pl.ds(sid * RPT, RPT)])
            plsc.subcore_barrier()

            def outer(j0, carry):
                for b in range(5):
                    j = j0 * 5 + b
                    pltpu.make_async_copy(xh.at[rowv.at[j]], buf.at[b],
                                          gsem.at[b]).wait()
                    pltpu.async_copy(buf.at[b], acc.at[colv.at[j]],
                                     ssem.at[b], add=True)
                    p = (b + 3) % 5
                    jp = j + 3

                    @pl.when(jp < CPT)
                    def _():
                        @pl.when(jp >= 5)
                        def _():
                            pltpu.make_async_copy(
                                buf.at[p], acc.at[colv.at[jp - 5]],
                                ssem.at[p]).wait()
                        pltpu.async_copy(xh.at[rowv.at[jp]], buf.at[p],
                                         gsem.at[p])
                return carry

            lax.fori_loop(0, CPT // 5, outer, 0, unroll=False)
            for b in range(5):
                pltpu.make_async_copy(buf.at[b],
                                      acc.at[colv.at[CPT - 5 + b]],
                                      ssem.at[b]).wait()
            plsc.subcore_barrier()
            pltpu.sync_copy(acc.at[pl.ds(sid * RPT, RPT)],
                            out_hbm.at[cid].at[h].at[pl.ds(sid * RPT, RPT)])

    return agg


_agg1 = _make_agg(2)
_agg2 = _make_agg(1)


def _tc1_body(deg_ref, x_ref, w1_ref, x1_ref, dv_ref):
    deg = jnp.sum(deg_ref[...], axis=0)[:, None] + 1.0
    dv = lax.rsqrt(deg)
    dv_ref[...] = dv
    x1 = jnp.dot(x_ref[...] * dv, w1_ref[...],
                 preferred_element_type=jnp.float32)
    x1_ref[0] = x1[:, :DP]
    x1_ref[1] = x1[:, DP:]


def _tc2_body(p_ref, x1_ref, dv_ref, b1_ref, w2_ref, x2_ref):
    s0 = p_ref[0, 0] + p_ref[1, 0] - x1_ref[0]
    s1 = p_ref[0, 1] + p_ref[1, 1] - x1_ref[1]
    s = jnp.concatenate([s0, s1], axis=1)
    dv = dv_ref[...]
    h = jnp.maximum(dv * s + b1_ref[...], 0.0)
    x2_ref[0] = jnp.dot(h, w2_ref[...],
                        preferred_element_type=jnp.float32) * dv


def _tc3_body(p_ref, x2_ref, dv_ref, b2_ref, out_ref):
    s = p_ref[0, 0] + p_ref[1, 0] - x2_ref[0]
    out_ref[...] = dv_ref[...] * s + b2_ref[...]


def _tc1(deg_t, x_pad, w1):
    return pl.pallas_call(
        _tc1_body,
        grid=(NP // BLK,),
        in_specs=[
            pl.BlockSpec((NW, BLK), lambda i: (0, i)),
            pl.BlockSpec((BLK, D_IN), lambda i: (i, 0)),
            pl.BlockSpec((D_IN, D_HID), lambda i: (0, 0)),
        ],
        out_specs=[
            pl.BlockSpec((2, BLK, DP), lambda i: (0, i, 0)),
            pl.BlockSpec((BLK, 1), lambda i: (i, 0)),
        ],
        out_shape=[
            jax.ShapeDtypeStruct((2, NP, DP), jnp.float32),
            jax.ShapeDtypeStruct((NP, 1), jnp.float32),
        ],
    )(deg_t, x_pad, w1)


def _tc2(p1, x1, dv, b1r, w2p):
    return pl.pallas_call(
        _tc2_body,
        grid=(NP // BLK,),
        in_specs=[
            pl.BlockSpec((NC, 2, BLK, DP), lambda i: (0, 0, i, 0)),
            pl.BlockSpec((2, BLK, DP), lambda i: (0, i, 0)),
            pl.BlockSpec((BLK, 1), lambda i: (i, 0)),
            pl.BlockSpec((1, D_HID), lambda i: (0, 0)),
            pl.BlockSpec((D_HID, DP), lambda i: (0, 0)),
        ],
        out_specs=pl.BlockSpec((1, BLK, DP), lambda i: (0, i, 0)),
        out_shape=jax.ShapeDtypeStruct((1, NP, DP), jnp.float32),
    )(p1, x1, dv, b1r, w2p)


def _tc3(p2, x2, dv, b2r):
    return pl.pallas_call(
        _tc3_body,
        grid=(NP // BLK,),
        in_specs=[
            pl.BlockSpec((NC, 1, BLK, DP), lambda i: (0, 0, i, 0)),
            pl.BlockSpec((1, BLK, DP), lambda i: (0, i, 0)),
            pl.BlockSpec((BLK, 1), lambda i: (i, 0)),
            pl.BlockSpec((1, DP), lambda i: (0, 0)),
        ],
        out_specs=pl.BlockSpec((BLK, DP), lambda i: (i, 0)),
        out_shape=jax.ShapeDtypeStruct((NP, DP), jnp.float32),
    )(p2, x2, dv, b2r)


def kernel(x, edge_index, W1, b1, W2, b2):
    ei = edge_index.astype(jnp.int32)
    npad = NE_PAD - N_EDGES
    it = lax.iota(jnp.int32, npad)
    rowp = jnp.concatenate([ei[0], it % N_NODES])
    colp = jnp.concatenate([ei[1], N_NODES + it % (NP - N_NODES)])
    row3 = rowp.reshape(NW, CPT, CH)
    col3 = colp.reshape(NW, CPT, CH)
    col2 = colp.reshape(NW, EPW)
    zeros = jnp.zeros((NP,), jnp.float32)
    x_pad = jnp.pad(x, ((0, NP - N_NODES), (0, 0)))
    b1r = b1.reshape(1, D_HID)
    w2p = jnp.pad(W2, ((0, 0), (0, DP - D_OUT)))
    b2r = jnp.pad(b2, (0, DP - D_OUT)).reshape(1, DP)

    deg = _deg_kernel(col2, zeros)
    x1, dv = _tc1(deg.reshape(NW, NP), x_pad, W1)
    p1 = _agg1(x1, row3, col3)
    x2 = _tc2(p1, x1, dv, b1r, w2p)
    p2 = _agg2(x2, row3, col3)
    out = _tc3(p2, x2, dv, b2r)
    return out[:N_NODES, :D_OUT]
